# baseline (speedup 1.0000x reference)
import functools
import jax
import jax.numpy as jnp
from jax import lax
from jax.experimental import pallas as pl
from jax.experimental.pallas import tpu as pltpu
from jax.experimental.pallas import tpu_sc as plsc

NC = 2
NS = 16
NW = NC * NS
L = 16

SUB = 80
SUBS_PER_BIG = 5
BIG = SUB * SUBS_PER_BIG

ACC_PAD_ROWS = 25088

BCH = 2272
BCAP = 25680
CAP = 26000

_params = pltpu.CompilerParams(needs_layout_passes=False,
                               use_tc_tiling_on_sc=False)


def _bucket_body(n_half, epw, bsrc, bdst, bw, osrc, odst, ow, cnts,
                 inb, inbw, cb, csrc, cdst, cw, sem):
    c = lax.axis_index("c")
    s = lax.axis_index("s")
    wid = s * NC + c
    e_base = wid * epw
    iota = lax.iota(jnp.int32, L)
    nfull = (epw // L) * L
    nch = nfull // BCH

    for g in range(2):
        lo = jnp.int32(g * n_half)
        cnt = jnp.int32(0)
        for ch in range(nch + 1):
            chn = BCH if ch < nch else (epw - nfull)
            off = e_base + ch * BCH
            pltpu.sync_copy(bsrc.at[pl.ds(off, chn)], inb.at[pl.ds(0, chn)])
            pltpu.sync_copy(bdst.at[pl.ds(off, chn)],
                            inb.at[pl.ds(BCH + 16, chn)])
            pltpu.sync_copy(bw.at[pl.ds(off, chn)], inbw.at[pl.ds(0, chn)])

            def compact(q, cnt, tail_mask=None):
                sv = inb[pl.ds(q * L, L)]
                dv = inb[pl.ds(BCH + 16 + q * L, L)]
                wv = inbw[pl.ds(q * L, L)]
                m = (dv >= lo) & (dv < lo + n_half)
                if tail_mask is not None:
                    m = m & tail_mask
                plsc.store_compressed(csrc.at[pl.ds(cnt, L)], sv, mask=m)
                plsc.store_compressed(cdst.at[pl.ds(cnt, L)], dv - lo, mask=m)
                plsc.store_compressed(cw.at[pl.ds(cnt, L)], wv, mask=m)
                return cnt + plsc.all_reduce_population_count(m)[0]

            if chn == BCH:
                cnt = lax.fori_loop(0, BCH // L, compact, cnt)
            else:
                cnt = compact(0, cnt, tail_mask=(iota < chn))

        zi = jnp.zeros((L,), jnp.int32)
        zf = jnp.zeros((L,), jnp.float32)
        for k in range(BIG // L):
            idx = cnt + iota + k * L
            plsc.store_scatter(csrc, [idx], zi)
            plsc.store_scatter(cdst, [idx], zi)
            plsc.store_scatter(cw, [idx], zf)
        cnt_pad = ((cnt + BIG - 1) // BIG) * BIG

        lbase = (g * NW + wid) * CAP
        pltpu.sync_copy(csrc, osrc.at[pl.ds(lbase, BCAP)])
        pltpu.sync_copy(cdst, odst.at[pl.ds(lbase, BCAP)])
        pltpu.sync_copy(cw, ow.at[pl.ds(lbase, BCAP)])

        cb[pl.ds(0, L)] = jnp.where(iota == 0, cnt_pad, 0)
        pltpu.sync_copy(cb, cnts.at[pl.ds((g * NW + wid) * L, L)])


def _make_bucket(n, e):
    n_half = n // NC
    epw = e // NW
    mesh = plsc.VectorSubcoreMesh(core_axis_name="c", subcore_axis_name="s")
    return pl.kernel(
        functools.partial(_bucket_body, n_half, epw),
        out_type=(
            jax.ShapeDtypeStruct((2 * NW * CAP,), jnp.int32),
            jax.ShapeDtypeStruct((2 * NW * CAP,), jnp.int32),
            jax.ShapeDtypeStruct((2 * NW * CAP,), jnp.float32),
            jax.ShapeDtypeStruct((2 * NW * L,), jnp.int32),
        ),
        mesh=mesh,
        scratch_types=[
            pltpu.VMEM((2 * (BCH + 16),), jnp.int32),
            pltpu.VMEM((BCH + 16,), jnp.float32),
            pltpu.VMEM((L,), jnp.int32),
            pltpu.VMEM((BCAP,), jnp.int32),
            pltpu.VMEM((BCAP,), jnp.int32),
            pltpu.VMEM((BCAP,), jnp.float32),
            pltpu.SemaphoreType.DMA,
        ],
        compiler_params=_params,
        name="lightgcn_bucket",
    )


def _propagate_body(n_half, emb_in, osrc, odst, ow, cnts, emb_out,
                    srcb, dstb, wb, dst2, cb,
                    rows0, rows1, rows2, rows3, rows4, acc,
                    gsem0, gsem1, gsem2, gsem3, gsem4,
                    ssem0, ssem1, ssem2, ssem3, ssem4):
    c = lax.axis_index("c")
    s = lax.axis_index("s")
    rows_bufs = (rows0, rows1, rows2, rows3, rows4)
    gsems = (gsem0, gsem1, gsem2, gsem3, gsem4)
    ssems = (ssem0, ssem1, ssem2, ssem3, ssem4)

    zvec = jnp.zeros((L,), jnp.float32)

    def zb_body(i, carry):
        for rows in rows_bufs:
            for r in range(4):
                rows[i, pl.ds(r * L, L)] = zvec
        return carry
    lax.fori_loop(0, SUB, zb_body, None)
    zivec = jnp.zeros((L,), jnp.int32)
    for j in range(SUBS_PER_BIG):
        for q in range(SUB // L):
            dst2[j, pl.ds(q * L, L)] = zivec

    rows_per_tile = ACC_PAD_ROWS // NS
    for k in range(rows_per_tile // SUB):
        pltpu.sync_copy(rows0, acc.at[pl.ds(s * rows_per_tile + k * SUB, SUB)])
    zrem = rows_per_tile % SUB
    if zrem:
        pltpu.sync_copy(
            rows0.at[pl.ds(0, zrem)],
            acc.at[pl.ds(s * rows_per_tile + rows_per_tile - zrem, zrem)])
    plsc.subcore_barrier()

    for j in range(SUBS_PER_BIG):
        pltpu.async_copy(rows_bufs[j], acc.at[dst2.at[j]], ssems[j], add=True)

    for k in range(2):
        lid = c * NW + (2 * s + k)
        pltpu.sync_copy(cnts.at[pl.ds(lid * L, L)], cb)
        nbigs = cb[pl.ds(0, L)][0] // BIG
        lbase = lid * CAP

        def big_body(g, carry):
            e_off = lbase + g * BIG
            pltpu.sync_copy(osrc.at[pl.ds(e_off, BIG)], srcb)
            for j in range(SUBS_PER_BIG):
                pltpu.make_async_copy(
                    rows_bufs[j], acc.at[dst2.at[j]], ssems[j]).wait()
                pltpu.async_copy(emb_in.at[srcb.at[pl.ds(j * SUB, SUB)]],
                                 rows_bufs[j], gsems[j])
            for j in range(SUBS_PER_BIG):
                pltpu.sync_copy(odst.at[pl.ds(e_off + j * SUB, SUB)],
                                dst2.at[j])
            pltpu.sync_copy(ow.at[pl.ds(e_off, BIG)], wb)

            for j in range(SUBS_PER_BIG):
                pltpu.make_async_copy(
                    emb_in.at[srcb.at[pl.ds(j * SUB, SUB)]],
                    rows_bufs[j], gsems[j]).wait()
                rows = rows_bufs[j]

                def scale_body(i, carry2):
                    ebase = jnp.full((L,), j * SUB, jnp.int32) + i * L
                    for kk in range(L):
                        wk = plsc.load_gather(wb, [ebase + kk])
                        e = i * L + kk
                        for r in range(4):
                            rows[e, pl.ds(r * L, L)] = (
                                rows[e, pl.ds(r * L, L)] * wk)
                    return carry2
                lax.fori_loop(0, SUB // L, scale_body, None)
                pltpu.async_copy(rows, acc.at[dst2.at[j]], ssems[j], add=True)
            return carry

        lax.fori_loop(0, nbigs, big_body, None)

    for j in range(SUBS_PER_BIG):
        pltpu.make_async_copy(rows_bufs[j], acc.at[dst2.at[j]], ssems[j]).wait()
    plsc.subcore_barrier()

    base = c * n_half
    wrows = 1560
    pltpu.sync_copy(acc.at[pl.ds(s * wrows, wrows)],
                    emb_out.at[pl.ds(base + s * wrows, wrows)])
    rem = n_half - NS * wrows
    if rem:
        @pl.when(s == NS - 1)
        def _tail():
            pltpu.sync_copy(acc.at[pl.ds(NS * wrows, rem)],
                            emb_out.at[pl.ds(base + NS * wrows, rem)])


def _make_propagate(n, d):
    n_half = n // NC
    assert d == 64
    mesh = plsc.VectorSubcoreMesh(core_axis_name="c", subcore_axis_name="s")
    return pl.kernel(
        functools.partial(_propagate_body, n_half),
        out_type=jax.ShapeDtypeStruct((n, d), jnp.float32),
        mesh=mesh,
        scratch_types=[
            pltpu.VMEM((BIG,), jnp.int32),
            pltpu.VMEM((BIG,), jnp.int32),
            pltpu.VMEM((BIG,), jnp.float32),
            pltpu.VMEM((SUBS_PER_BIG, SUB), jnp.int32),
            pltpu.VMEM((L,), jnp.int32),
            pltpu.VMEM((SUB, 64), jnp.float32),
            pltpu.VMEM((SUB, 64), jnp.float32),
            pltpu.VMEM((SUB, 64), jnp.float32),
            pltpu.VMEM((SUB, 64), jnp.float32),
            pltpu.VMEM((SUB, 64), jnp.float32),
            pltpu.VMEM_SHARED((ACC_PAD_ROWS, 64), jnp.float32),
        ] + [pltpu.SemaphoreType.DMA] * 10,
        compiler_params=_params,
        name="lightgcn_propagate",
    )


def _final_body(n_users, bpt, users, items, emb0, emb3, gamma,
                ub, ib, u0r, u3r, i0r, i3r, gb, sem):
    c = lax.axis_index("c")
    s = lax.axis_index("s")
    wid = s * NC + c
    b0 = wid * bpt

    pltpu.sync_copy(users.at[pl.ds(b0, bpt)], ub)
    pltpu.sync_copy(items.at[pl.ds(b0, bpt)], ib)
    for q in range(bpt // L):
        ib[pl.ds(q * L, L)] = ib[pl.ds(q * L, L)] + jnp.int32(n_users)

    pltpu.async_copy(emb0.at[ub], u0r, sem).wait()
    pltpu.async_copy(emb3.at[ub], u3r, sem).wait()
    pltpu.async_copy(emb0.at[ib], i0r, sem).wait()
    pltpu.async_copy(emb3.at[ib], i3r, sem).wait()

    iota = lax.iota(jnp.int32, L)
    for q in range(bpt // L):
        bvec = iota + jnp.int32(q * L)

        def dot_body(dd, acc):
            dvec = jnp.full((L,), dd, jnp.int32)
            u0 = plsc.load_gather(u0r, [bvec, dvec])
            u3 = plsc.load_gather(u3r, [bvec, dvec])
            i0 = plsc.load_gather(i0r, [bvec, dvec])
            i3 = plsc.load_gather(i3r, [bvec, dvec])
            return acc + (u0 + u3) * (i0 + i3)
        acc = lax.fori_loop(0, 64, dot_body, jnp.zeros((L,), jnp.float32))
        gb[pl.ds(q * L, L)] = acc * 0.25

    pltpu.sync_copy(gb, gamma.at[pl.ds(b0, bpt)])


def _make_final(n_users, b):
    bpt = b // NW
    mesh = plsc.VectorSubcoreMesh(core_axis_name="c", subcore_axis_name="s")
    return pl.kernel(
        functools.partial(_final_body, n_users, bpt),
        out_type=jax.ShapeDtypeStruct((b,), jnp.float32),
        mesh=mesh,
        scratch_types=[
            pltpu.VMEM((bpt,), jnp.int32),
            pltpu.VMEM((bpt,), jnp.int32),
            pltpu.VMEM((bpt, 64), jnp.float32),
            pltpu.VMEM((bpt, 64), jnp.float32),
            pltpu.VMEM((bpt, 64), jnp.float32),
            pltpu.VMEM((bpt, 64), jnp.float32),
            pltpu.VMEM((bpt,), jnp.float32),
            pltpu.SemaphoreType.DMA,
        ],
        compiler_params=_params,
        name="lightgcn_final",
    )


@jax.jit
def kernel(users, items, edge_index, edge_values, user_emb, item_emb):
    n_users, d = user_emb.shape
    n = n_users + item_emb.shape[0]
    e = edge_values.shape[0]
    b = users.shape[0]

    emb0 = jnp.concatenate([user_emb, item_emb], axis=0)

    osrc, odst, ow, cnts = _make_bucket(n, e)(
        edge_index[0], edge_index[1], edge_values)
    propagate = _make_propagate(n, d)
    emb = emb0
    for _ in range(3):
        emb = propagate(emb, osrc, odst, ow, cnts)
    return _make_final(n_users, b)(users, items, emb0, emb)


# speedup vs baseline: 1.0656x; 1.0656x over previous
import functools
import jax
import jax.numpy as jnp
from jax import lax
from jax.experimental import pallas as pl
from jax.experimental.pallas import tpu as pltpu
from jax.experimental.pallas import tpu_sc as plsc

NC = 2
NS = 16
NW = NC * NS
L = 16

SUB = 80
SUBS_PER_BIG = 10
NRB = 5
BIG = SUB * SUBS_PER_BIG

ACC_PAD_ROWS = 25088

BCH = 2272
BCAP = 25840
CAP = 26400

_params = pltpu.CompilerParams(needs_layout_passes=False,
                               use_tc_tiling_on_sc=False)


def _bucket_body(n_half, epw, bsrc, bdst, bw, osrc, odst, ow, cnts,
                 inb, inbw, cb, csrc, cdst, cw, sem):
    c = lax.axis_index("c")
    s = lax.axis_index("s")
    wid = s * NC + c
    e_base = wid * epw
    iota = lax.iota(jnp.int32, L)
    nfull = (epw // L) * L
    nch = nfull // BCH

    for g in range(2):
        lo = jnp.int32(g * n_half)
        cnt = jnp.int32(0)
        for ch in range(nch + 1):
            chn = BCH if ch < nch else (epw - nfull)
            off = e_base + ch * BCH
            pltpu.sync_copy(bsrc.at[pl.ds(off, chn)], inb.at[pl.ds(0, chn)])
            pltpu.sync_copy(bdst.at[pl.ds(off, chn)],
                            inb.at[pl.ds(BCH + 16, chn)])
            pltpu.sync_copy(bw.at[pl.ds(off, chn)], inbw.at[pl.ds(0, chn)])

            def compact(q, cnt, tail_mask=None):
                sv = inb[pl.ds(q * L, L)]
                dv = inb[pl.ds(BCH + 16 + q * L, L)]
                wv = inbw[pl.ds(q * L, L)]
                m = (dv >= lo) & (dv < lo + n_half)
                if tail_mask is not None:
                    m = m & tail_mask
                plsc.store_compressed(csrc.at[pl.ds(cnt, L)], sv, mask=m)
                plsc.store_compressed(cdst.at[pl.ds(cnt, L)], dv - lo, mask=m)
                plsc.store_compressed(cw.at[pl.ds(cnt, L)], wv, mask=m)
                return cnt + plsc.all_reduce_population_count(m)[0]

            if chn == BCH:
                cnt = lax.fori_loop(0, BCH // L, compact, cnt)
            else:
                cnt = compact(0, cnt, tail_mask=(iota < chn))

        zi = jnp.zeros((L,), jnp.int32)
        zf = jnp.zeros((L,), jnp.float32)
        for k in range(BIG // L):
            idx = cnt + iota + k * L
            plsc.store_scatter(csrc, [idx], zi)
            plsc.store_scatter(cdst, [idx], zi)
            plsc.store_scatter(cw, [idx], zf)
        cnt_pad = ((cnt + BIG - 1) // BIG) * BIG

        lbase = (g * NW + wid) * CAP
        pltpu.sync_copy(csrc, osrc.at[pl.ds(lbase, BCAP)])
        pltpu.sync_copy(cdst, odst.at[pl.ds(lbase, BCAP)])
        pltpu.sync_copy(cw, ow.at[pl.ds(lbase, BCAP)])

        cb[pl.ds(0, L)] = jnp.where(iota == 0, cnt_pad, 0)
        pltpu.sync_copy(cb, cnts.at[pl.ds((g * NW + wid) * L, L)])


def _make_bucket(n, e):
    n_half = n // NC
    epw = e // NW
    mesh = plsc.VectorSubcoreMesh(core_axis_name="c", subcore_axis_name="s")
    return pl.kernel(
        functools.partial(_bucket_body, n_half, epw),
        out_type=(
            jax.ShapeDtypeStruct((2 * NW * CAP,), jnp.int32),
            jax.ShapeDtypeStruct((2 * NW * CAP,), jnp.int32),
            jax.ShapeDtypeStruct((2 * NW * CAP,), jnp.float32),
            jax.ShapeDtypeStruct((2 * NW * L,), jnp.int32),
        ),
        mesh=mesh,
        scratch_types=[
            pltpu.VMEM((2 * (BCH + 16),), jnp.int32),
            pltpu.VMEM((BCH + 16,), jnp.float32),
            pltpu.VMEM((L,), jnp.int32),
            pltpu.VMEM((BCAP,), jnp.int32),
            pltpu.VMEM((BCAP,), jnp.int32),
            pltpu.VMEM((BCAP,), jnp.float32),
            pltpu.SemaphoreType.DMA,
        ],
        compiler_params=_params,
        name="lightgcn_bucket",
    )


def _propagate_body(n_half, emb_in, osrc, odst, ow, cnts, emb_out,
                    srcb, dstb, wb, dst2, cb,
                    rows0, rows1, rows2, rows3, rows4, acc,
                    gsem0, gsem1, gsem2, gsem3, gsem4,
                    ssem0, ssem1, ssem2, ssem3, ssem4):
    c = lax.axis_index("c")
    s = lax.axis_index("s")
    rows_bufs = (rows0, rows1, rows2, rows3, rows4)
    gsems = (gsem0, gsem1, gsem2, gsem3, gsem4)
    ssems = (ssem0, ssem1, ssem2, ssem3, ssem4)

    zvec = jnp.zeros((L,), jnp.float32)

    def zb_body(i, carry):
        for rows in rows_bufs:
            for r in range(4):
                rows[i, pl.ds(r * L, L)] = zvec
        return carry
    lax.fori_loop(0, SUB, zb_body, None)
    zivec = jnp.zeros((L,), jnp.int32)
    for j in range(SUBS_PER_BIG):
        for q in range(SUB // L):
            dst2[j, pl.ds(q * L, L)] = zivec

    rows_per_tile = ACC_PAD_ROWS // NS
    for k in range(rows_per_tile // SUB):
        pltpu.sync_copy(rows0, acc.at[pl.ds(s * rows_per_tile + k * SUB, SUB)])
    zrem = rows_per_tile % SUB
    if zrem:
        pltpu.sync_copy(
            rows0.at[pl.ds(0, zrem)],
            acc.at[pl.ds(s * rows_per_tile + rows_per_tile - zrem, zrem)])
    plsc.subcore_barrier()

    for j in range(NRB):
        pltpu.async_copy(rows_bufs[j], acc.at[dst2.at[j]], ssems[j], add=True)

    for k in range(2):
        lid = c * NW + (2 * s + k)
        pltpu.sync_copy(cnts.at[pl.ds(lid * L, L)], cb)
        nbigs = cb[pl.ds(0, L)][0] // BIG
        lbase = lid * CAP

        def big_body(g, carry):
            e_off = lbase + g * BIG
            pltpu.sync_copy(osrc.at[pl.ds(e_off, BIG)], srcb)
            for j in range(NRB):
                pltpu.make_async_copy(
                    rows_bufs[j], acc.at[dst2.at[j]], ssems[j]).wait()
                pltpu.async_copy(emb_in.at[srcb.at[pl.ds(j * SUB, SUB)]],
                                 rows_bufs[j], gsems[j])
            pltpu.sync_copy(odst.at[pl.ds(e_off, BIG)], dstb)
            pltpu.sync_copy(ow.at[pl.ds(e_off, BIG)], wb)

            for j in range(SUBS_PER_BIG):
                for q in range(SUB // L):
                    dst2[j, pl.ds(q * L, L)] = dstb[pl.ds(j * SUB + q * L, L)]

            for j in range(SUBS_PER_BIG):
                p = j % NRB
                pltpu.make_async_copy(
                    emb_in.at[srcb.at[pl.ds(j * SUB, SUB)]],
                    rows_bufs[p], gsems[p]).wait()
                rows = rows_bufs[p]

                def scale_body(i, carry2):
                    ebase = jnp.full((L,), j * SUB, jnp.int32) + i * L
                    for kk in range(L):
                        wk = plsc.load_gather(wb, [ebase + kk])
                        e = i * L + kk
                        for r in range(4):
                            rows[e, pl.ds(r * L, L)] = (
                                rows[e, pl.ds(r * L, L)] * wk)
                    return carry2
                lax.fori_loop(0, SUB // L, scale_body, None)
                pltpu.async_copy(rows, acc.at[dst2.at[j]], ssems[p], add=True)
                if j + NRB < SUBS_PER_BIG:
                    pltpu.make_async_copy(
                        rows, acc.at[dst2.at[j]], ssems[p]).wait()
                    pltpu.async_copy(
                        emb_in.at[srcb.at[pl.ds((j + NRB) * SUB, SUB)]],
                        rows_bufs[p], gsems[p])
            return carry

        lax.fori_loop(0, nbigs, big_body, None)

    for j in range(NRB):
        pltpu.make_async_copy(rows_bufs[j], acc.at[dst2.at[j]], ssems[j]).wait()
    plsc.subcore_barrier()

    base = c * n_half
    wrows = 1560
    pltpu.sync_copy(acc.at[pl.ds(s * wrows, wrows)],
                    emb_out.at[pl.ds(base + s * wrows, wrows)])
    rem = n_half - NS * wrows
    if rem:
        @pl.when(s == NS - 1)
        def _tail():
            pltpu.sync_copy(acc.at[pl.ds(NS * wrows, rem)],
                            emb_out.at[pl.ds(base + NS * wrows, rem)])


def _make_propagate(n, d):
    n_half = n // NC
    assert d == 64
    mesh = plsc.VectorSubcoreMesh(core_axis_name="c", subcore_axis_name="s")
    return pl.kernel(
        functools.partial(_propagate_body, n_half),
        out_type=jax.ShapeDtypeStruct((n, d), jnp.float32),
        mesh=mesh,
        scratch_types=[
            pltpu.VMEM((BIG,), jnp.int32),
            pltpu.VMEM((BIG,), jnp.int32),
            pltpu.VMEM((BIG,), jnp.float32),
            pltpu.VMEM((SUBS_PER_BIG, SUB), jnp.int32),
            pltpu.VMEM((L,), jnp.int32),
            pltpu.VMEM((SUB, 64), jnp.float32),
            pltpu.VMEM((SUB, 64), jnp.float32),
            pltpu.VMEM((SUB, 64), jnp.float32),
            pltpu.VMEM((SUB, 64), jnp.float32),
            pltpu.VMEM((SUB, 64), jnp.float32),
            pltpu.VMEM_SHARED((ACC_PAD_ROWS, 64), jnp.float32),
        ] + [pltpu.SemaphoreType.DMA] * 10,
        compiler_params=_params,
        name="lightgcn_propagate",
    )


def _final_body(n_users, bpt, users, items, emb0, emb3, gamma,
                ub, ib, u0r, u3r, i0r, i3r, gb, sem):
    c = lax.axis_index("c")
    s = lax.axis_index("s")
    wid = s * NC + c
    b0 = wid * bpt

    pltpu.sync_copy(users.at[pl.ds(b0, bpt)], ub)
    pltpu.sync_copy(items.at[pl.ds(b0, bpt)], ib)
    for q in range(bpt // L):
        ib[pl.ds(q * L, L)] = ib[pl.ds(q * L, L)] + jnp.int32(n_users)

    pltpu.async_copy(emb0.at[ub], u0r, sem).wait()
    pltpu.async_copy(emb3.at[ub], u3r, sem).wait()
    pltpu.async_copy(emb0.at[ib], i0r, sem).wait()
    pltpu.async_copy(emb3.at[ib], i3r, sem).wait()

    iota = lax.iota(jnp.int32, L)
    for q in range(bpt // L):
        bvec = iota + jnp.int32(q * L)

        def dot_body(dd, acc):
            dvec = jnp.full((L,), dd, jnp.int32)
            u0 = plsc.load_gather(u0r, [bvec, dvec])
            u3 = plsc.load_gather(u3r, [bvec, dvec])
            i0 = plsc.load_gather(i0r, [bvec, dvec])
            i3 = plsc.load_gather(i3r, [bvec, dvec])
            return acc + (u0 + u3) * (i0 + i3)
        acc = lax.fori_loop(0, 64, dot_body, jnp.zeros((L,), jnp.float32))
        gb[pl.ds(q * L, L)] = acc * 0.25

    pltpu.sync_copy(gb, gamma.at[pl.ds(b0, bpt)])


def _make_final(n_users, b):
    bpt = b // NW
    mesh = plsc.VectorSubcoreMesh(core_axis_name="c", subcore_axis_name="s")
    return pl.kernel(
        functools.partial(_final_body, n_users, bpt),
        out_type=jax.ShapeDtypeStruct((b,), jnp.float32),
        mesh=mesh,
        scratch_types=[
            pltpu.VMEM((bpt,), jnp.int32),
            pltpu.VMEM((bpt,), jnp.int32),
            pltpu.VMEM((bpt, 64), jnp.float32),
            pltpu.VMEM((bpt, 64), jnp.float32),
            pltpu.VMEM((bpt, 64), jnp.float32),
            pltpu.VMEM((bpt, 64), jnp.float32),
            pltpu.VMEM((bpt,), jnp.float32),
            pltpu.SemaphoreType.DMA,
        ],
        compiler_params=_params,
        name="lightgcn_final",
    )


@jax.jit
def kernel(users, items, edge_index, edge_values, user_emb, item_emb):
    n_users, d = user_emb.shape
    n = n_users + item_emb.shape[0]
    e = edge_values.shape[0]
    b = users.shape[0]

    emb0 = jnp.concatenate([user_emb, item_emb], axis=0)

    osrc, odst, ow, cnts = _make_bucket(n, e)(
        edge_index[0], edge_index[1], edge_values)
    propagate = _make_propagate(n, d)
    emb = emb0
    for _ in range(3):
        emb = propagate(emb, osrc, odst, ow, cnts)
    return _make_final(n_users, b)(users, items, emb0, emb)


# speedup vs baseline: 1.1526x; 1.0817x over previous
import functools
import jax
import jax.numpy as jnp
from jax import lax
from jax.experimental import pallas as pl
from jax.experimental.pallas import tpu as pltpu
from jax.experimental.pallas import tpu_sc as plsc

NC = 2
NS = 16
NW = NC * NS
L = 16

SUB = 80
SUBS_PER_BIG = 5
NRB = 5
BIG = SUB * SUBS_PER_BIG

ACC_PAD_ROWS = 25088

BCH = 2272
BCAP = 25680
CAP = 26000

_params = pltpu.CompilerParams(needs_layout_passes=False,
                               use_tc_tiling_on_sc=False)


def _bucket_body(n_half, epw, bsrc, bdst, bw, osrc, odst, ow, cnts,
                 inb, inbw, cb, csrc, cdst, cw, sem):
    c = lax.axis_index("c")
    s = lax.axis_index("s")
    wid = s * NC + c
    e_base = wid * epw
    iota = lax.iota(jnp.int32, L)
    nfull = (epw // L) * L
    nch = nfull // BCH

    for g in range(2):
        lo = jnp.int32(g * n_half)
        cnt = jnp.int32(0)
        for ch in range(nch + 1):
            chn = BCH if ch < nch else (epw - nfull)
            off = e_base + ch * BCH
            pltpu.sync_copy(bsrc.at[pl.ds(off, chn)], inb.at[pl.ds(0, chn)])
            pltpu.sync_copy(bdst.at[pl.ds(off, chn)],
                            inb.at[pl.ds(BCH + 16, chn)])
            pltpu.sync_copy(bw.at[pl.ds(off, chn)], inbw.at[pl.ds(0, chn)])

            def compact(q, cnt, tail_mask=None):
                sv = inb[pl.ds(q * L, L)]
                dv = inb[pl.ds(BCH + 16 + q * L, L)]
                wv = inbw[pl.ds(q * L, L)]
                m = (dv >= lo) & (dv < lo + n_half)
                if tail_mask is not None:
                    m = m & tail_mask
                plsc.store_compressed(csrc.at[pl.ds(cnt, L)], sv, mask=m)
                plsc.store_compressed(cdst.at[pl.ds(cnt, L)], dv - lo, mask=m)
                plsc.store_compressed(cw.at[pl.ds(cnt, L)], wv, mask=m)
                return cnt + plsc.all_reduce_population_count(m)[0]

            if chn == BCH:
                cnt = lax.fori_loop(0, BCH // L, compact, cnt)
            else:
                cnt = compact(0, cnt, tail_mask=(iota < chn))

        zi = jnp.zeros((L,), jnp.int32)
        zf = jnp.zeros((L,), jnp.float32)
        for k in range(BIG // L):
            idx = cnt + iota + k * L
            plsc.store_scatter(csrc, [idx], zi)
            plsc.store_scatter(cdst, [idx], zi)
            plsc.store_scatter(cw, [idx], zf)
        cnt_pad = ((cnt + BIG - 1) // BIG) * BIG

        lbase = (g * NW + wid) * CAP
        pltpu.sync_copy(csrc, osrc.at[pl.ds(lbase, BCAP)])
        pltpu.sync_copy(cdst, odst.at[pl.ds(lbase, BCAP)])
        pltpu.sync_copy(cw, ow.at[pl.ds(lbase, BCAP)])

        cb[pl.ds(0, L)] = jnp.where(iota == 0, cnt_pad, 0)
        pltpu.sync_copy(cb, cnts.at[pl.ds((g * NW + wid) * L, L)])


def _make_bucket(n, e):
    n_half = n // NC
    epw = e // NW
    mesh = plsc.VectorSubcoreMesh(core_axis_name="c", subcore_axis_name="s")
    return pl.kernel(
        functools.partial(_bucket_body, n_half, epw),
        out_type=(
            jax.ShapeDtypeStruct((2 * NW * CAP,), jnp.int32),
            jax.ShapeDtypeStruct((2 * NW * CAP,), jnp.int32),
            jax.ShapeDtypeStruct((2 * NW * CAP,), jnp.float32),
            jax.ShapeDtypeStruct((2 * NW * L,), jnp.int32),
        ),
        mesh=mesh,
        scratch_types=[
            pltpu.VMEM((2 * (BCH + 16),), jnp.int32),
            pltpu.VMEM((BCH + 16,), jnp.float32),
            pltpu.VMEM((L,), jnp.int32),
            pltpu.VMEM((BCAP,), jnp.int32),
            pltpu.VMEM((BCAP,), jnp.int32),
            pltpu.VMEM((BCAP,), jnp.float32),
            pltpu.SemaphoreType.DMA,
        ],
        compiler_params=_params,
        name="lightgcn_bucket",
    )


def _propagate_body(n_half, emb_in, osrc, odst, ow, cnts, emb_out,
                    srcb, dstb, wb, dst2, cb,
                    rows0, rows1, rows2, rows3, rows4, acc,
                    gsem0, gsem1, gsem2, gsem3, gsem4,
                    ssem0, ssem1, ssem2, ssem3, ssem4, esem0, esem1):
    c = lax.axis_index("c")
    s = lax.axis_index("s")
    rows_bufs = (rows0, rows1, rows2, rows3, rows4)
    gsems = (gsem0, gsem1, gsem2, gsem3, gsem4)
    ssems = (ssem0, ssem1, ssem2, ssem3, ssem4)

    zvec = jnp.zeros((L,), jnp.float32)

    def zb_body(i, carry):
        for rows in rows_bufs:
            for r in range(4):
                rows[i, pl.ds(r * L, L)] = zvec
        return carry
    lax.fori_loop(0, SUB, zb_body, None)
    zivec = jnp.zeros((L,), jnp.int32)
    for j in range(SUBS_PER_BIG):
        for q in range(SUB // L):
            dst2[j, pl.ds(q * L, L)] = zivec

    rows_per_tile = ACC_PAD_ROWS // NS
    for k in range(rows_per_tile // SUB):
        pltpu.sync_copy(rows0, acc.at[pl.ds(s * rows_per_tile + k * SUB, SUB)])
    zrem = rows_per_tile % SUB
    if zrem:
        pltpu.sync_copy(
            rows0.at[pl.ds(0, zrem)],
            acc.at[pl.ds(s * rows_per_tile + rows_per_tile - zrem, zrem)])
    plsc.subcore_barrier()

    for j in range(NRB):
        pltpu.async_copy(rows_bufs[j], acc.at[dst2.at[j]], ssems[j], add=True)

    for k in range(2):
        lid = c * NW + (2 * s + k)
        pltpu.sync_copy(cnts.at[pl.ds(lid * L, L)], cb)
        nbigs = cb[pl.ds(0, L)][0] // BIG
        lbase = lid * CAP

        def big_body(g, carry):
            e_off = lbase + g * BIG
            pltpu.sync_copy(osrc.at[pl.ds(e_off, BIG)], srcb)
            pltpu.async_copy(odst.at[pl.ds(e_off, BIG)], dstb, esem0)
            pltpu.async_copy(ow.at[pl.ds(e_off, BIG)], wb, esem1)
            for j in range(NRB):
                pltpu.make_async_copy(
                    rows_bufs[j], acc.at[dst2.at[j]], ssems[j]).wait()
                pltpu.async_copy(emb_in.at[srcb.at[pl.ds(j * SUB, SUB)]],
                                 rows_bufs[j], gsems[j])
            pltpu.make_async_copy(
                odst.at[pl.ds(e_off, BIG)], dstb, esem0).wait()
            pltpu.make_async_copy(ow.at[pl.ds(e_off, BIG)], wb, esem1).wait()

            for j in range(SUBS_PER_BIG):
                for q in range(SUB // L):
                    dst2[j, pl.ds(q * L, L)] = dstb[pl.ds(j * SUB + q * L, L)]

            for j in range(SUBS_PER_BIG):
                p = j % NRB
                pltpu.make_async_copy(
                    emb_in.at[srcb.at[pl.ds(j * SUB, SUB)]],
                    rows_bufs[p], gsems[p]).wait()
                rows = rows_bufs[p]

                def scale_body(i, carry2):
                    ebase = jnp.full((L,), j * SUB, jnp.int32) + i * L
                    for kk in range(L):
                        wk = plsc.load_gather(wb, [ebase + kk])
                        e = i * L + kk
                        for r in range(4):
                            rows[e, pl.ds(r * L, L)] = (
                                rows[e, pl.ds(r * L, L)] * wk)
                    return carry2
                lax.fori_loop(0, SUB // L, scale_body, None)
                pltpu.async_copy(rows, acc.at[dst2.at[j]], ssems[p], add=True)
                if j + NRB < SUBS_PER_BIG:
                    pltpu.make_async_copy(
                        rows, acc.at[dst2.at[j]], ssems[p]).wait()
                    pltpu.async_copy(
                        emb_in.at[srcb.at[pl.ds((j + NRB) * SUB, SUB)]],
                        rows_bufs[p], gsems[p])
            return carry

        lax.fori_loop(0, nbigs, big_body, None)

    for j in range(NRB):
        pltpu.make_async_copy(rows_bufs[j], acc.at[dst2.at[j]], ssems[j]).wait()
    plsc.subcore_barrier()

    base = c * n_half
    wrows = 1560
    pltpu.sync_copy(acc.at[pl.ds(s * wrows, wrows)],
                    emb_out.at[pl.ds(base + s * wrows, wrows)])
    rem = n_half - NS * wrows
    if rem:
        @pl.when(s == NS - 1)
        def _tail():
            pltpu.sync_copy(acc.at[pl.ds(NS * wrows, rem)],
                            emb_out.at[pl.ds(base + NS * wrows, rem)])


def _make_propagate(n, d):
    n_half = n // NC
    assert d == 64
    mesh = plsc.VectorSubcoreMesh(core_axis_name="c", subcore_axis_name="s")
    return pl.kernel(
        functools.partial(_propagate_body, n_half),
        out_type=jax.ShapeDtypeStruct((n, d), jnp.float32),
        mesh=mesh,
        scratch_types=[
            pltpu.VMEM((BIG,), jnp.int32),
            pltpu.VMEM((BIG,), jnp.int32),
            pltpu.VMEM((BIG,), jnp.float32),
            pltpu.VMEM((SUBS_PER_BIG, SUB), jnp.int32),
            pltpu.VMEM((L,), jnp.int32),
            pltpu.VMEM((SUB, 64), jnp.float32),
            pltpu.VMEM((SUB, 64), jnp.float32),
            pltpu.VMEM((SUB, 64), jnp.float32),
            pltpu.VMEM((SUB, 64), jnp.float32),
            pltpu.VMEM((SUB, 64), jnp.float32),
            pltpu.VMEM_SHARED((ACC_PAD_ROWS, 64), jnp.float32),
        ] + [pltpu.SemaphoreType.DMA] * 12,
        compiler_params=_params,
        name="lightgcn_propagate",
    )


def _final_body(n_users, bpt, users, items, emb0, emb3, gamma,
                ub, ib, u0r, u3r, i0r, i3r, gb, sem):
    c = lax.axis_index("c")
    s = lax.axis_index("s")
    wid = s * NC + c
    b0 = wid * bpt

    pltpu.sync_copy(users.at[pl.ds(b0, bpt)], ub)
    pltpu.sync_copy(items.at[pl.ds(b0, bpt)], ib)
    for q in range(bpt // L):
        ib[pl.ds(q * L, L)] = ib[pl.ds(q * L, L)] + jnp.int32(n_users)

    pltpu.async_copy(emb0.at[ub], u0r, sem).wait()
    pltpu.async_copy(emb3.at[ub], u3r, sem).wait()
    pltpu.async_copy(emb0.at[ib], i0r, sem).wait()
    pltpu.async_copy(emb3.at[ib], i3r, sem).wait()

    iota = lax.iota(jnp.int32, L)
    for q in range(bpt // L):
        bvec = iota + jnp.int32(q * L)

        def dot_body(dd, acc):
            dvec = jnp.full((L,), dd, jnp.int32)
            u0 = plsc.load_gather(u0r, [bvec, dvec])
            u3 = plsc.load_gather(u3r, [bvec, dvec])
            i0 = plsc.load_gather(i0r, [bvec, dvec])
            i3 = plsc.load_gather(i3r, [bvec, dvec])
            return acc + (u0 + u3) * (i0 + i3)
        acc = lax.fori_loop(0, 64, dot_body, jnp.zeros((L,), jnp.float32))
        gb[pl.ds(q * L, L)] = acc * 0.25

    pltpu.sync_copy(gb, gamma.at[pl.ds(b0, bpt)])


def _make_final(n_users, b):
    bpt = b // NW
    mesh = plsc.VectorSubcoreMesh(core_axis_name="c", subcore_axis_name="s")
    return pl.kernel(
        functools.partial(_final_body, n_users, bpt),
        out_type=jax.ShapeDtypeStruct((b,), jnp.float32),
        mesh=mesh,
        scratch_types=[
            pltpu.VMEM((bpt,), jnp.int32),
            pltpu.VMEM((bpt,), jnp.int32),
            pltpu.VMEM((bpt, 64), jnp.float32),
            pltpu.VMEM((bpt, 64), jnp.float32),
            pltpu.VMEM((bpt, 64), jnp.float32),
            pltpu.VMEM((bpt, 64), jnp.float32),
            pltpu.VMEM((bpt,), jnp.float32),
            pltpu.SemaphoreType.DMA,
        ],
        compiler_params=_params,
        name="lightgcn_final",
    )


@jax.jit
def kernel(users, items, edge_index, edge_values, user_emb, item_emb):
    n_users, d = user_emb.shape
    n = n_users + item_emb.shape[0]
    e = edge_values.shape[0]
    b = users.shape[0]

    emb0 = jnp.concatenate([user_emb, item_emb], axis=0)

    osrc, odst, ow, cnts = _make_bucket(n, e)(
        edge_index[0], edge_index[1], edge_values)
    propagate = _make_propagate(n, d)
    emb = emb0
    for _ in range(3):
        emb = propagate(emb, osrc, odst, ow, cnts)
    return _make_final(n_users, b)(users, items, emb0, emb)


# speedup vs baseline: 1.1631x; 1.0091x over previous
import functools
import jax
import jax.numpy as jnp
from jax import lax
from jax.experimental import pallas as pl
from jax.experimental.pallas import tpu as pltpu
from jax.experimental.pallas import tpu_sc as plsc

NC = 2
NS = 16
NW = NC * NS
L = 16

SUB = 80
SUBS_PER_BIG = 5
NRB = 5
BIG = SUB * SUBS_PER_BIG

ACC_PAD_ROWS = 25088

BCH = 2272
BCAP = 25680
CAP = 26000

_params = pltpu.CompilerParams(needs_layout_passes=False,
                               use_tc_tiling_on_sc=False)


def _bucket_body(n_half, epw, bsrc, bdst, bw, osrc, odst, ow, cnts,
                 inb, inbw, cb, csrc, cdst, cw, sem):
    c = lax.axis_index("c")
    s = lax.axis_index("s")
    wid = s * NC + c
    e_base = wid * epw
    iota = lax.iota(jnp.int32, L)
    nfull = (epw // L) * L
    nch = nfull // BCH

    for g in range(2):
        lo = jnp.int32(g * n_half)
        cnt = jnp.int32(0)
        for ch in range(nch + 1):
            chn = BCH if ch < nch else (epw - nfull)
            off = e_base + ch * BCH
            pltpu.sync_copy(bsrc.at[pl.ds(off, chn)], inb.at[pl.ds(0, chn)])
            pltpu.sync_copy(bdst.at[pl.ds(off, chn)],
                            inb.at[pl.ds(BCH + 16, chn)])
            pltpu.sync_copy(bw.at[pl.ds(off, chn)], inbw.at[pl.ds(0, chn)])

            def compact(q, cnt, tail_mask=None):
                sv = inb[pl.ds(q * L, L)]
                dv = inb[pl.ds(BCH + 16 + q * L, L)]
                wv = inbw[pl.ds(q * L, L)]
                m = (dv >= lo) & (dv < lo + n_half)
                if tail_mask is not None:
                    m = m & tail_mask
                plsc.store_compressed(csrc.at[pl.ds(cnt, L)], sv, mask=m)
                plsc.store_compressed(cdst.at[pl.ds(cnt, L)], dv - lo, mask=m)
                plsc.store_compressed(cw.at[pl.ds(cnt, L)], wv, mask=m)
                return cnt + plsc.all_reduce_population_count(m)[0]

            if chn == BCH:
                cnt = lax.fori_loop(0, BCH // L, compact, cnt)
            else:
                cnt = compact(0, cnt, tail_mask=(iota < chn))

        zi = jnp.zeros((L,), jnp.int32)
        zf = jnp.zeros((L,), jnp.float32)
        for k in range(BIG // L):
            idx = cnt + iota + k * L
            plsc.store_scatter(csrc, [idx], zi)
            plsc.store_scatter(cdst, [idx], zi)
            plsc.store_scatter(cw, [idx], zf)
        cnt_pad = ((cnt + BIG - 1) // BIG) * BIG

        lbase = (g * NW + wid) * CAP
        pltpu.sync_copy(csrc, osrc.at[pl.ds(lbase, BCAP)])
        pltpu.sync_copy(cdst, odst.at[pl.ds(lbase, BCAP)])
        pltpu.sync_copy(cw, ow.at[pl.ds(lbase, BCAP)])

        cb[pl.ds(0, L)] = jnp.where(iota == 0, cnt_pad, 0)
        pltpu.sync_copy(cb, cnts.at[pl.ds((g * NW + wid) * L, L)])


def _make_bucket(n, e):
    n_half = n // NC
    epw = e // NW
    mesh = plsc.VectorSubcoreMesh(core_axis_name="c", subcore_axis_name="s")
    return pl.kernel(
        functools.partial(_bucket_body, n_half, epw),
        out_type=(
            jax.ShapeDtypeStruct((2 * NW * CAP,), jnp.int32),
            jax.ShapeDtypeStruct((2 * NW * CAP,), jnp.int32),
            jax.ShapeDtypeStruct((2 * NW * CAP,), jnp.float32),
            jax.ShapeDtypeStruct((2 * NW * L,), jnp.int32),
        ),
        mesh=mesh,
        scratch_types=[
            pltpu.VMEM((2 * (BCH + 16),), jnp.int32),
            pltpu.VMEM((BCH + 16,), jnp.float32),
            pltpu.VMEM((L,), jnp.int32),
            pltpu.VMEM((BCAP,), jnp.int32),
            pltpu.VMEM((BCAP,), jnp.int32),
            pltpu.VMEM((BCAP,), jnp.float32),
            pltpu.SemaphoreType.DMA,
        ],
        compiler_params=_params,
        name="lightgcn_bucket",
    )


def _propagate_body(n_half, emb_in, osrc, odst, ow, cnts, emb_out,
                    srcb, dstb, wb, dst2, cb,
                    rows0, rows1, rows2, rows3, rows4, acc,
                    gsem0, gsem1, gsem2, gsem3, gsem4,
                    ssem0, ssem1, ssem2, ssem3, ssem4, esem0, esem1, esem2):
    c = lax.axis_index("c")
    s = lax.axis_index("s")
    rows_bufs = (rows0, rows1, rows2, rows3, rows4)
    gsems = (gsem0, gsem1, gsem2, gsem3, gsem4)
    ssems = (ssem0, ssem1, ssem2, ssem3, ssem4)

    zvec = jnp.zeros((L,), jnp.float32)

    def zb_body(i, carry):
        for rows in rows_bufs:
            for r in range(4):
                rows[i, pl.ds(r * L, L)] = zvec
        return carry
    lax.fori_loop(0, SUB, zb_body, None)
    zivec = jnp.zeros((L,), jnp.int32)
    for j in range(SUBS_PER_BIG):
        for q in range(SUB // L):
            dst2[j, pl.ds(q * L, L)] = zivec

    rows_per_tile = ACC_PAD_ROWS // NS
    nz = rows_per_tile // SUB
    for k in range(nz):
        pltpu.async_copy(rows0, acc.at[pl.ds(s * rows_per_tile + k * SUB, SUB)],
                         esem0)
    zrem = rows_per_tile % SUB
    if zrem:
        pltpu.async_copy(
            rows0.at[pl.ds(0, zrem)],
            acc.at[pl.ds(s * rows_per_tile + rows_per_tile - zrem, zrem)],
            esem0)
    for k in range(nz):
        pltpu.make_async_copy(
            rows0, acc.at[pl.ds(s * rows_per_tile + k * SUB, SUB)],
            esem0).wait()
    if zrem:
        pltpu.make_async_copy(
            rows0.at[pl.ds(0, zrem)],
            acc.at[pl.ds(s * rows_per_tile + rows_per_tile - zrem, zrem)],
            esem0).wait()
    plsc.subcore_barrier()

    for j in range(NRB):
        pltpu.async_copy(rows_bufs[j], acc.at[dst2.at[j]], ssems[j], add=True)

    for k in range(2):
        lid = c * NW + (2 * s + k)
        pltpu.sync_copy(cnts.at[pl.ds(lid * L, L)], cb)
        nbigs = cb[pl.ds(0, L)][0] // BIG
        lbase = lid * CAP

        def big_body(g, carry):
            e_off = lbase + g * BIG
            pltpu.async_copy(osrc.at[pl.ds(e_off, BIG)], srcb, esem2)
            pltpu.async_copy(odst.at[pl.ds(e_off, BIG)], dstb, esem0)
            pltpu.async_copy(ow.at[pl.ds(e_off, BIG)], wb, esem1)
            for j in range(NRB):
                pltpu.make_async_copy(
                    rows_bufs[j], acc.at[dst2.at[j]], ssems[j]).wait()
            pltpu.make_async_copy(
                osrc.at[pl.ds(e_off, BIG)], srcb, esem2).wait()
            for j in range(NRB):
                pltpu.async_copy(emb_in.at[srcb.at[pl.ds(j * SUB, SUB)]],
                                 rows_bufs[j], gsems[j])
            pltpu.make_async_copy(
                odst.at[pl.ds(e_off, BIG)], dstb, esem0).wait()
            pltpu.make_async_copy(ow.at[pl.ds(e_off, BIG)], wb, esem1).wait()

            for j in range(SUBS_PER_BIG):
                for q in range(SUB // L):
                    dst2[j, pl.ds(q * L, L)] = dstb[pl.ds(j * SUB + q * L, L)]

            for j in range(SUBS_PER_BIG):
                p = j % NRB
                pltpu.make_async_copy(
                    emb_in.at[srcb.at[pl.ds(j * SUB, SUB)]],
                    rows_bufs[p], gsems[p]).wait()
                rows = rows_bufs[p]

                def scale_body(i, carry2):
                    ebase = jnp.full((L,), j * SUB, jnp.int32) + i * L
                    for kk in range(L):
                        wk = plsc.load_gather(wb, [ebase + kk])
                        e = i * L + kk
                        for r in range(4):
                            rows[e, pl.ds(r * L, L)] = (
                                rows[e, pl.ds(r * L, L)] * wk)
                    return carry2
                lax.fori_loop(0, SUB // L, scale_body, None)
                pltpu.async_copy(rows, acc.at[dst2.at[j]], ssems[p], add=True)
                if j + NRB < SUBS_PER_BIG:
                    pltpu.make_async_copy(
                        rows, acc.at[dst2.at[j]], ssems[p]).wait()
                    pltpu.async_copy(
                        emb_in.at[srcb.at[pl.ds((j + NRB) * SUB, SUB)]],
                        rows_bufs[p], gsems[p])
            return carry

        lax.fori_loop(0, nbigs, big_body, None)

    for j in range(NRB):
        pltpu.make_async_copy(rows_bufs[j], acc.at[dst2.at[j]], ssems[j]).wait()
    plsc.subcore_barrier()

    base = c * n_half
    wrows = 1560
    pltpu.sync_copy(acc.at[pl.ds(s * wrows, wrows)],
                    emb_out.at[pl.ds(base + s * wrows, wrows)])
    rem = n_half - NS * wrows
    if rem:
        @pl.when(s == NS - 1)
        def _tail():
            pltpu.sync_copy(acc.at[pl.ds(NS * wrows, rem)],
                            emb_out.at[pl.ds(base + NS * wrows, rem)])


def _make_propagate(n, d):
    n_half = n // NC
    assert d == 64
    mesh = plsc.VectorSubcoreMesh(core_axis_name="c", subcore_axis_name="s")
    return pl.kernel(
        functools.partial(_propagate_body, n_half),
        out_type=jax.ShapeDtypeStruct((n, d), jnp.float32),
        mesh=mesh,
        scratch_types=[
            pltpu.VMEM((BIG,), jnp.int32),
            pltpu.VMEM((BIG,), jnp.int32),
            pltpu.VMEM((BIG,), jnp.float32),
            pltpu.VMEM((SUBS_PER_BIG, SUB), jnp.int32),
            pltpu.VMEM((L,), jnp.int32),
            pltpu.VMEM((SUB, 64), jnp.float32),
            pltpu.VMEM((SUB, 64), jnp.float32),
            pltpu.VMEM((SUB, 64), jnp.float32),
            pltpu.VMEM((SUB, 64), jnp.float32),
            pltpu.VMEM((SUB, 64), jnp.float32),
            pltpu.VMEM_SHARED((ACC_PAD_ROWS, 64), jnp.float32),
        ] + [pltpu.SemaphoreType.DMA] * 13,
        compiler_params=_params,
        name="lightgcn_propagate",
    )


def _final_body(n_users, bpt, users, items, emb0, emb3, gamma,
                ub, ib, u0r, u3r, i0r, i3r, gb, sem):
    c = lax.axis_index("c")
    s = lax.axis_index("s")
    wid = s * NC + c
    b0 = wid * bpt

    pltpu.sync_copy(users.at[pl.ds(b0, bpt)], ub)
    pltpu.sync_copy(items.at[pl.ds(b0, bpt)], ib)
    for q in range(bpt // L):
        ib[pl.ds(q * L, L)] = ib[pl.ds(q * L, L)] + jnp.int32(n_users)

    pltpu.async_copy(emb0.at[ub], u0r, sem).wait()
    pltpu.async_copy(emb3.at[ub], u3r, sem).wait()
    pltpu.async_copy(emb0.at[ib], i0r, sem).wait()
    pltpu.async_copy(emb3.at[ib], i3r, sem).wait()

    iota = lax.iota(jnp.int32, L)
    for q in range(bpt // L):
        bvec = iota + jnp.int32(q * L)

        def dot_body(dd, acc):
            dvec = jnp.full((L,), dd, jnp.int32)
            u0 = plsc.load_gather(u0r, [bvec, dvec])
            u3 = plsc.load_gather(u3r, [bvec, dvec])
            i0 = plsc.load_gather(i0r, [bvec, dvec])
            i3 = plsc.load_gather(i3r, [bvec, dvec])
            return acc + (u0 + u3) * (i0 + i3)
        acc = lax.fori_loop(0, 64, dot_body, jnp.zeros((L,), jnp.float32))
        gb[pl.ds(q * L, L)] = acc * 0.25

    pltpu.sync_copy(gb, gamma.at[pl.ds(b0, bpt)])


def _make_final(n_users, b):
    bpt = b // NW
    mesh = plsc.VectorSubcoreMesh(core_axis_name="c", subcore_axis_name="s")
    return pl.kernel(
        functools.partial(_final_body, n_users, bpt),
        out_type=jax.ShapeDtypeStruct((b,), jnp.float32),
        mesh=mesh,
        scratch_types=[
            pltpu.VMEM((bpt,), jnp.int32),
            pltpu.VMEM((bpt,), jnp.int32),
            pltpu.VMEM((bpt, 64), jnp.float32),
            pltpu.VMEM((bpt, 64), jnp.float32),
            pltpu.VMEM((bpt, 64), jnp.float32),
            pltpu.VMEM((bpt, 64), jnp.float32),
            pltpu.VMEM((bpt,), jnp.float32),
            pltpu.SemaphoreType.DMA,
        ],
        compiler_params=_params,
        name="lightgcn_final",
    )


@jax.jit
def kernel(users, items, edge_index, edge_values, user_emb, item_emb):
    n_users, d = user_emb.shape
    n = n_users + item_emb.shape[0]
    e = edge_values.shape[0]
    b = users.shape[0]

    emb0 = jnp.concatenate([user_emb, item_emb], axis=0)

    osrc, odst, ow, cnts = _make_bucket(n, e)(
        edge_index[0], edge_index[1], edge_values)
    propagate = _make_propagate(n, d)
    emb = emb0
    for _ in range(3):
        emb = propagate(emb, osrc, odst, ow, cnts)
    return _make_final(n_users, b)(users, items, emb0, emb)


# speedup vs baseline: 1.2922x; 1.1110x over previous
import functools
import jax
import jax.numpy as jnp
from jax import lax
from jax.experimental import pallas as pl
from jax.experimental.pallas import tpu as pltpu
from jax.experimental.pallas import tpu_sc as plsc

NC = 2
NS = 16
NW = NC * NS
L = 16

SUB = 80
SUBS_PER_BIG = 5
NRB = 5
BIG = SUB * SUBS_PER_BIG

ACC_PAD_ROWS = 25088

BCH = 2272
BCAP = 25680
CAP = 26000

_params = pltpu.CompilerParams(needs_layout_passes=False,
                               use_tc_tiling_on_sc=False)


def _bucket_body(n_half, epw, bsrc, bdst, bw, osrc, odst, ow, cnts,
                 inb, inbw, cb, csrc, cdst, cw, sem):
    c = lax.axis_index("c")
    s = lax.axis_index("s")
    wid = s * NC + c
    e_base = wid * epw
    iota = lax.iota(jnp.int32, L)
    nfull = (epw // L) * L
    nch = nfull // BCH

    for g in range(2):
        lo = jnp.int32(g * n_half)
        cnt = jnp.int32(0)
        for ch in range(nch + 1):
            chn = BCH if ch < nch else (epw - nfull)
            off = e_base + ch * BCH
            pltpu.sync_copy(bsrc.at[pl.ds(off, chn)], inb.at[pl.ds(0, chn)])
            pltpu.sync_copy(bdst.at[pl.ds(off, chn)],
                            inb.at[pl.ds(BCH + 16, chn)])
            pltpu.sync_copy(bw.at[pl.ds(off, chn)], inbw.at[pl.ds(0, chn)])

            def compact(q, cnt, tail_mask=None):
                sv = inb[pl.ds(q * L, L)]
                dv = inb[pl.ds(BCH + 16 + q * L, L)]
                wv = inbw[pl.ds(q * L, L)]
                m = (dv >= lo) & (dv < lo + n_half)
                if tail_mask is not None:
                    m = m & tail_mask
                plsc.store_compressed(csrc.at[pl.ds(cnt, L)], sv, mask=m)
                plsc.store_compressed(cdst.at[pl.ds(cnt, L)], dv - lo, mask=m)
                plsc.store_compressed(cw.at[pl.ds(cnt, L)], wv, mask=m)
                return cnt + plsc.all_reduce_population_count(m)[0]

            if chn == BCH:
                cnt = lax.fori_loop(0, BCH // L, compact, cnt)
            else:
                cnt = compact(0, cnt, tail_mask=(iota < chn))

        zi = jnp.zeros((L,), jnp.int32)
        zf = jnp.zeros((L,), jnp.float32)
        for k in range(BIG // L):
            idx = cnt + iota + k * L
            plsc.store_scatter(csrc, [idx], zi)
            plsc.store_scatter(cdst, [idx], zi)
            plsc.store_scatter(cw, [idx], zf)
        cnt_pad = ((cnt + BIG - 1) // BIG) * BIG

        lbase = (g * NW + wid) * CAP
        pltpu.sync_copy(csrc, osrc.at[pl.ds(lbase, BCAP)])
        pltpu.sync_copy(cdst, odst.at[pl.ds(lbase, BCAP)])
        pltpu.sync_copy(cw, ow.at[pl.ds(lbase, BCAP)])

        cb[pl.ds(0, L)] = jnp.where(iota == 0, cnt_pad, 0)
        pltpu.sync_copy(cb, cnts.at[pl.ds((g * NW + wid) * L, L)])


def _make_bucket(n, e):
    n_half = n // NC
    epw = e // NW
    mesh = plsc.VectorSubcoreMesh(core_axis_name="c", subcore_axis_name="s")
    return pl.kernel(
        functools.partial(_bucket_body, n_half, epw),
        out_type=(
            jax.ShapeDtypeStruct((2 * NW * CAP,), jnp.int32),
            jax.ShapeDtypeStruct((2 * NW * CAP,), jnp.int32),
            jax.ShapeDtypeStruct((2 * NW * CAP,), jnp.float32),
            jax.ShapeDtypeStruct((2 * NW * L,), jnp.int32),
        ),
        mesh=mesh,
        scratch_types=[
            pltpu.VMEM((2 * (BCH + 16),), jnp.int32),
            pltpu.VMEM((BCH + 16,), jnp.float32),
            pltpu.VMEM((L,), jnp.int32),
            pltpu.VMEM((BCAP,), jnp.int32),
            pltpu.VMEM((BCAP,), jnp.int32),
            pltpu.VMEM((BCAP,), jnp.float32),
            pltpu.SemaphoreType.DMA,
        ],
        compiler_params=_params,
        name="lightgcn_bucket",
    )


def _propagate_body(n_half, emb_in, osrc, odst, ow, cnts, emb_out,
                    srcb, dstb, wb, dst2, cb,
                    rows0, rows1, rows2, rows3, rows4, acc,
                    gsem0, gsem1, gsem2, gsem3, gsem4,
                    ssem0, ssem1, ssem2, ssem3, ssem4, esem0, esem1, esem2):
    c = lax.axis_index("c")
    s = lax.axis_index("s")
    rows_bufs = (rows0, rows1, rows2, rows3, rows4)
    gsems = (gsem0, gsem1, gsem2, gsem3, gsem4)
    ssems = (ssem0, ssem1, ssem2, ssem3, ssem4)

    zvec = jnp.zeros((L,), jnp.float32)

    def zb_body(i, carry):
        for rows in rows_bufs:
            for r in range(4):
                rows[i, pl.ds(r * L, L)] = zvec
        return carry
    lax.fori_loop(0, SUB, zb_body, None)
    zivec = jnp.zeros((L,), jnp.int32)
    for j in range(SUBS_PER_BIG):
        for q in range(SUB // L):
            dst2[j, pl.ds(q * L, L)] = zivec

    rows_per_tile = ACC_PAD_ROWS // NS
    nz = rows_per_tile // SUB
    for k in range(nz):
        pltpu.async_copy(rows0, acc.at[pl.ds(s * rows_per_tile + k * SUB, SUB)],
                         esem0)
    zrem = rows_per_tile % SUB
    if zrem:
        pltpu.async_copy(
            rows0.at[pl.ds(0, zrem)],
            acc.at[pl.ds(s * rows_per_tile + rows_per_tile - zrem, zrem)],
            esem0)
    for k in range(nz):
        pltpu.make_async_copy(
            rows0, acc.at[pl.ds(s * rows_per_tile + k * SUB, SUB)],
            esem0).wait()
    if zrem:
        pltpu.make_async_copy(
            rows0.at[pl.ds(0, zrem)],
            acc.at[pl.ds(s * rows_per_tile + rows_per_tile - zrem, zrem)],
            esem0).wait()
    plsc.subcore_barrier()

    for j in range(NRB):
        pltpu.async_copy(rows_bufs[j], acc.at[dst2.at[j]], ssems[j], add=True)

    for k in range(2):
        lid = c * NW + (2 * s + k)
        pltpu.sync_copy(cnts.at[pl.ds(lid * L, L)], cb)
        nbigs = cb[pl.ds(0, L)][0] // BIG
        lbase = lid * CAP

        def big_body(g, carry):
            e_off = lbase + g * BIG
            pltpu.async_copy(osrc.at[pl.ds(e_off, BIG)], srcb, esem2)
            pltpu.async_copy(odst.at[pl.ds(e_off, BIG)], dstb, esem0)
            pltpu.async_copy(ow.at[pl.ds(e_off, BIG)], wb, esem1)
            for j in range(NRB):
                pltpu.make_async_copy(
                    rows_bufs[j], acc.at[dst2.at[j]], ssems[j]).wait()
            pltpu.make_async_copy(
                osrc.at[pl.ds(e_off, BIG)], srcb, esem2).wait()
            for j in range(NRB):
                pltpu.async_copy(emb_in.at[srcb.at[pl.ds(j * SUB, SUB)]],
                                 rows_bufs[j], gsems[j])
            pltpu.make_async_copy(
                odst.at[pl.ds(e_off, BIG)], dstb, esem0).wait()
            pltpu.make_async_copy(ow.at[pl.ds(e_off, BIG)], wb, esem1).wait()

            for j in range(SUBS_PER_BIG):
                for q in range(SUB // L):
                    dst2[j, pl.ds(q * L, L)] = dstb[pl.ds(j * SUB + q * L, L)]

            for j in range(SUBS_PER_BIG):
                p = j % NRB
                pltpu.make_async_copy(
                    emb_in.at[srcb.at[pl.ds(j * SUB, SUB)]],
                    rows_bufs[p], gsems[p]).wait()
                rows = rows_bufs[p]

                @plsc.parallel_loop(0, SUB // L)
                def scale_body(i):
                    ebase = jnp.full((L,), j * SUB, jnp.int32) + i * L
                    for kk in range(L):
                        wk = plsc.load_gather(wb, [ebase + kk])
                        e = i * L + kk
                        for r in range(4):
                            rows[e, pl.ds(r * L, L)] = (
                                rows[e, pl.ds(r * L, L)] * wk)
                pltpu.async_copy(rows, acc.at[dst2.at[j]], ssems[p], add=True)
                if j + NRB < SUBS_PER_BIG:
                    pltpu.make_async_copy(
                        rows, acc.at[dst2.at[j]], ssems[p]).wait()
                    pltpu.async_copy(
                        emb_in.at[srcb.at[pl.ds((j + NRB) * SUB, SUB)]],
                        rows_bufs[p], gsems[p])
            return carry

        lax.fori_loop(0, nbigs, big_body, None)

    for j in range(NRB):
        pltpu.make_async_copy(rows_bufs[j], acc.at[dst2.at[j]], ssems[j]).wait()
    plsc.subcore_barrier()

    base = c * n_half
    wrows = 1560
    pltpu.sync_copy(acc.at[pl.ds(s * wrows, wrows)],
                    emb_out.at[pl.ds(base + s * wrows, wrows)])
    rem = n_half - NS * wrows
    if rem:
        @pl.when(s == NS - 1)
        def _tail():
            pltpu.sync_copy(acc.at[pl.ds(NS * wrows, rem)],
                            emb_out.at[pl.ds(base + NS * wrows, rem)])


def _make_propagate(n, d):
    n_half = n // NC
    assert d == 64
    mesh = plsc.VectorSubcoreMesh(core_axis_name="c", subcore_axis_name="s")
    return pl.kernel(
        functools.partial(_propagate_body, n_half),
        out_type=jax.ShapeDtypeStruct((n, d), jnp.float32),
        mesh=mesh,
        scratch_types=[
            pltpu.VMEM((BIG,), jnp.int32),
            pltpu.VMEM((BIG,), jnp.int32),
            pltpu.VMEM((BIG,), jnp.float32),
            pltpu.VMEM((SUBS_PER_BIG, SUB), jnp.int32),
            pltpu.VMEM((L,), jnp.int32),
            pltpu.VMEM((SUB, 64), jnp.float32),
            pltpu.VMEM((SUB, 64), jnp.float32),
            pltpu.VMEM((SUB, 64), jnp.float32),
            pltpu.VMEM((SUB, 64), jnp.float32),
            pltpu.VMEM((SUB, 64), jnp.float32),
            pltpu.VMEM_SHARED((ACC_PAD_ROWS, 64), jnp.float32),
        ] + [pltpu.SemaphoreType.DMA] * 13,
        compiler_params=_params,
        name="lightgcn_propagate",
    )


def _final_body(n_users, bpt, users, items, emb0, emb3, gamma,
                ub, ib, u0r, u3r, i0r, i3r, gb, sem):
    c = lax.axis_index("c")
    s = lax.axis_index("s")
    wid = s * NC + c
    b0 = wid * bpt

    pltpu.sync_copy(users.at[pl.ds(b0, bpt)], ub)
    pltpu.sync_copy(items.at[pl.ds(b0, bpt)], ib)
    for q in range(bpt // L):
        ib[pl.ds(q * L, L)] = ib[pl.ds(q * L, L)] + jnp.int32(n_users)

    pltpu.async_copy(emb0.at[ub], u0r, sem).wait()
    pltpu.async_copy(emb3.at[ub], u3r, sem).wait()
    pltpu.async_copy(emb0.at[ib], i0r, sem).wait()
    pltpu.async_copy(emb3.at[ib], i3r, sem).wait()

    iota = lax.iota(jnp.int32, L)
    for q in range(bpt // L):
        bvec = iota + jnp.int32(q * L)

        def dot_body(dd, acc):
            dvec = jnp.full((L,), dd, jnp.int32)
            u0 = plsc.load_gather(u0r, [bvec, dvec])
            u3 = plsc.load_gather(u3r, [bvec, dvec])
            i0 = plsc.load_gather(i0r, [bvec, dvec])
            i3 = plsc.load_gather(i3r, [bvec, dvec])
            return acc + (u0 + u3) * (i0 + i3)
        acc = lax.fori_loop(0, 64, dot_body, jnp.zeros((L,), jnp.float32))
        gb[pl.ds(q * L, L)] = acc * 0.25

    pltpu.sync_copy(gb, gamma.at[pl.ds(b0, bpt)])


def _make_final(n_users, b):
    bpt = b // NW
    mesh = plsc.VectorSubcoreMesh(core_axis_name="c", subcore_axis_name="s")
    return pl.kernel(
        functools.partial(_final_body, n_users, bpt),
        out_type=jax.ShapeDtypeStruct((b,), jnp.float32),
        mesh=mesh,
        scratch_types=[
            pltpu.VMEM((bpt,), jnp.int32),
            pltpu.VMEM((bpt,), jnp.int32),
            pltpu.VMEM((bpt, 64), jnp.float32),
            pltpu.VMEM((bpt, 64), jnp.float32),
            pltpu.VMEM((bpt, 64), jnp.float32),
            pltpu.VMEM((bpt, 64), jnp.float32),
            pltpu.VMEM((bpt,), jnp.float32),
            pltpu.SemaphoreType.DMA,
        ],
        compiler_params=_params,
        name="lightgcn_final",
    )


@jax.jit
def kernel(users, items, edge_index, edge_values, user_emb, item_emb):
    n_users, d = user_emb.shape
    n = n_users + item_emb.shape[0]
    e = edge_values.shape[0]
    b = users.shape[0]

    emb0 = jnp.concatenate([user_emb, item_emb], axis=0)

    osrc, odst, ow, cnts = _make_bucket(n, e)(
        edge_index[0], edge_index[1], edge_values)
    propagate = _make_propagate(n, d)
    emb = emb0
    for _ in range(3):
        emb = propagate(emb, osrc, odst, ow, cnts)
    return _make_final(n_users, b)(users, items, emb0, emb)
